# baseline (speedup 1.0000x reference)
import functools

import jax
import jax.numpy as jnp
from jax import lax
from jax.experimental import pallas as pl
from jax.experimental.pallas import tpu as pltpu
from jax.experimental.pallas import tpu_sc as plsc

_NUM_PERSON = 1000000
_NUM_ITEM = 1000
_LATENT_DIM = 64
_BATCH = 16384


def _sc_gather(table128, row_idx):
    info = plsc.get_sparse_core_info()
    nc, ns = info.num_cores, info.num_subcores
    nw = nc * ns
    b_per_w = _BATCH // nw

    mesh = plsc.VectorSubcoreMesh(core_axis_name="c", subcore_axis_name="s")

    @functools.partial(
        pl.kernel,
        mesh=mesh,
        out_type=jax.ShapeDtypeStruct((_BATCH, 2 * _LATENT_DIM), jnp.float32),
        scratch_types=[
            pltpu.VMEM((b_per_w,), jnp.int32),
            pltpu.VMEM((b_per_w, 2 * _LATENT_DIM), jnp.float32),
            pltpu.SemaphoreType.DMA,
        ],
    )
    def gather_kernel(table_hbm, idx_hbm, out_hbm, idx_v, rows_v, sem):
        wid = lax.axis_index("s") * nc + lax.axis_index("c")
        base = wid * b_per_w
        pltpu.sync_copy(idx_hbm.at[pl.ds(base, b_per_w)], idx_v)
        pltpu.async_copy(table_hbm.at[idx_v], rows_v, sem).wait()
        pltpu.sync_copy(rows_v, out_hbm.at[pl.ds(base, b_per_w)])

    return gather_kernel(table128, row_idx)


def _tc_decode(gathered, parity, diff):
    bb = 1024
    grid = (_BATCH // bb,)

    def body(g_ref, p_ref, d_ref, o_ref):
        g = g_ref[...]
        s0 = jnp.sum(g[:, :_LATENT_DIM], axis=1, keepdims=True)
        s1 = jnp.sum(g[:, _LATENT_DIM:], axis=1, keepdims=True)
        s = jnp.where(p_ref[...] > 0, s1, s0)
        o_ref[...] = jax.nn.sigmoid(s + d_ref[...])

    return pl.pallas_call(
        body,
        grid=grid,
        in_specs=[
            pl.BlockSpec((bb, 2 * _LATENT_DIM), lambda i: (i, 0)),
            pl.BlockSpec((bb, 1), lambda i: (i, 0)),
            pl.BlockSpec((1, _NUM_ITEM), lambda i: (0, 0)),
        ],
        out_specs=pl.BlockSpec((bb, _NUM_ITEM), lambda i: (i, 0)),
        out_shape=jax.ShapeDtypeStruct((_BATCH, _NUM_ITEM), jnp.float32),
    )(gathered, parity, diff)


def kernel(index, response, mask, ability_table, item_table):
    idx = index.astype(jnp.int32)
    table128 = ability_table.reshape(_NUM_PERSON // 2, 2 * _LATENT_DIM)
    row_idx = lax.shift_right_logical(idx, 1)
    parity = (idx & 1).reshape(_BATCH, 1)
    gathered = _sc_gather(table128, row_idx)
    diff = item_table.reshape(1, _NUM_ITEM)
    out = _tc_decode(gathered, parity, diff)
    return out[..., None]


# speedup vs baseline: 4.7379x; 4.7379x over previous
import functools

import jax
import jax.numpy as jnp
from jax import lax
from jax.experimental import pallas as pl
from jax.experimental.pallas import tpu as pltpu
from jax.experimental.pallas import tpu_sc as plsc

_NUM_PERSON = 1000000
_NUM_ITEM = 1000
_LATENT_DIM = 64
_BATCH = 16384

_COLS = 16384
_NBLK = (_NUM_PERSON + _COLS - 1) // _COLS
_PAD_W = _NBLK * _COLS


def _tc_reduce(table_t):
    def body(x_ref, o_ref):
        o_ref[...] = jnp.sum(x_ref[...], axis=0, keepdims=True)

    return pl.pallas_call(
        body,
        grid=(_NBLK,),
        in_specs=[pl.BlockSpec((_LATENT_DIM, _COLS), lambda i: (0, i))],
        out_specs=pl.BlockSpec((1, _COLS), lambda i: (0, i)),
        out_shape=jax.ShapeDtypeStruct((1, _PAD_W), jnp.float32),
    )(table_t)


def _sc_gather(table, row_idx):
    info = plsc.get_sparse_core_info()
    nc, ns = info.num_cores, info.num_subcores
    nw = nc * ns
    b_per_w = _BATCH // nw

    mesh = plsc.VectorSubcoreMesh(core_axis_name="c", subcore_axis_name="s")

    @functools.partial(
        pl.kernel,
        mesh=mesh,
        out_type=jax.ShapeDtypeStruct((_BATCH, 128), jnp.float32),
        scratch_types=[
            pltpu.VMEM((b_per_w,), jnp.int32),
            pltpu.VMEM((b_per_w, 128), jnp.float32),
            pltpu.SemaphoreType.DMA,
        ],
    )
    def gather_kernel(table_hbm, idx_hbm, out_hbm, idx_v, rows_v, sem):
        wid = lax.axis_index("s") * nc + lax.axis_index("c")
        base = wid * b_per_w
        pltpu.sync_copy(idx_hbm.at[pl.ds(base, b_per_w)], idx_v)
        pltpu.async_copy(table_hbm.at[idx_v], rows_v, sem).wait()
        pltpu.sync_copy(rows_v, out_hbm.at[pl.ds(base, b_per_w)])

    return gather_kernel(table, row_idx)


def _tc_decode(gathered, lane, diff):
    bb = 1024
    grid = (_BATCH // bb,)

    def body(g_ref, l_ref, d_ref, o_ref):
        li = lax.broadcasted_iota(jnp.int32, (bb, 128), 1)
        sel = jnp.where(li == l_ref[...], g_ref[...], 0.0)
        s = jnp.sum(sel, axis=1, keepdims=True)
        x = d_ref[...] + s.T
        o_ref[...] = (0.5 * jnp.tanh(0.5 * x) + 0.5)[:, None, :]

    return pl.pallas_call(
        body,
        grid=grid,
        in_specs=[
            pl.BlockSpec((bb, 128), lambda i: (i, 0)),
            pl.BlockSpec((bb, 1), lambda i: (i, 0)),
            pl.BlockSpec((_NUM_ITEM, 1), lambda i: (0, 0)),
        ],
        out_specs=pl.BlockSpec((_NUM_ITEM, 1, bb), lambda i: (0, 0, i)),
        out_shape=jax.ShapeDtypeStruct((_NUM_ITEM, 1, _BATCH), jnp.float32),
    )(gathered, lane, diff)


def kernel(index, response, mask, ability_table, item_table):
    idx = index.astype(jnp.int32)
    table_t = ability_table.T
    sums = _tc_reduce(table_t)
    packed = sums.reshape(_PAD_W // 128, 128)
    row = lax.shift_right_logical(idx, 7)
    lane = (idx & 127).reshape(_BATCH, 1)
    gathered = _sc_gather(packed, row)
    out_t = _tc_decode(gathered, lane, item_table)
    return jnp.transpose(out_t, (2, 0, 1))


# speedup vs baseline: 5.1569x; 1.0884x over previous
import functools

import jax
import jax.numpy as jnp
from jax import lax
from jax.experimental import pallas as pl
from jax.experimental.pallas import tpu as pltpu
from jax.experimental.pallas import tpu_sc as plsc

_NUM_PERSON = 1000000
_NUM_ITEM = 1000
_LATENT_DIM = 64
_BATCH = 16384

_COLS = 32768
_NBLK = (_NUM_PERSON + _COLS - 1) // _COLS
_PAD_W = _NBLK * _COLS


def _tc_reduce(table_t):
    def body(x_ref, o_ref):
        o_ref[...] = jnp.sum(x_ref[...], axis=0, keepdims=True)

    return pl.pallas_call(
        body,
        grid=(_NBLK,),
        in_specs=[pl.BlockSpec((_LATENT_DIM, _COLS), lambda i: (0, i))],
        out_specs=pl.BlockSpec((1, _COLS), lambda i: (0, i)),
        out_shape=jax.ShapeDtypeStruct((1, _PAD_W), jnp.float32),
    )(table_t)


def _sc_gather(table, row_idx):
    info = plsc.get_sparse_core_info()
    nc, ns = info.num_cores, info.num_subcores
    nw = nc * ns
    b_per_w = _BATCH // nw

    mesh = plsc.VectorSubcoreMesh(core_axis_name="c", subcore_axis_name="s")

    @functools.partial(
        pl.kernel,
        mesh=mesh,
        out_type=jax.ShapeDtypeStruct((_BATCH, 128), jnp.float32),
        scratch_types=[
            pltpu.VMEM((b_per_w,), jnp.int32),
            pltpu.VMEM((b_per_w, 128), jnp.float32),
            pltpu.SemaphoreType.DMA,
        ],
    )
    def gather_kernel(table_hbm, idx_hbm, out_hbm, idx_v, rows_v, sem):
        wid = lax.axis_index("s") * nc + lax.axis_index("c")
        base = wid * b_per_w
        pltpu.sync_copy(idx_hbm.at[pl.ds(base, b_per_w)], idx_v)
        pltpu.async_copy(table_hbm.at[idx_v], rows_v, sem).wait()
        pltpu.sync_copy(rows_v, out_hbm.at[pl.ds(base, b_per_w)])

    return gather_kernel(table, row_idx)


def _tc_decode(gathered, lane, diff):
    bb = 2048
    grid = (_BATCH // bb,)

    def body(g_ref, l_ref, d_ref, o_ref):
        li = lax.broadcasted_iota(jnp.int32, (bb, 128), 1)
        sel = jnp.where(li == l_ref[...], g_ref[...], 0.0)
        s = 0.5 * jnp.sum(sel, axis=1, keepdims=True)
        x = 0.5 * d_ref[...] + s.T
        o_ref[...] = (0.5 * jnp.tanh(x) + 0.5)[:, None, :]

    return pl.pallas_call(
        body,
        grid=grid,
        in_specs=[
            pl.BlockSpec((bb, 128), lambda i: (i, 0)),
            pl.BlockSpec((bb, 1), lambda i: (i, 0)),
            pl.BlockSpec((_NUM_ITEM, 1), lambda i: (0, 0)),
        ],
        out_specs=pl.BlockSpec((_NUM_ITEM, 1, bb), lambda i: (0, 0, i)),
        out_shape=jax.ShapeDtypeStruct((_NUM_ITEM, 1, _BATCH), jnp.float32),
    )(gathered, lane, diff)


def kernel(index, response, mask, ability_table, item_table):
    idx = index.astype(jnp.int32)
    table_t = ability_table.T
    sums = _tc_reduce(table_t)
    packed = sums.reshape(_PAD_W // 128, 128)
    row = lax.shift_right_logical(idx, 7)
    lane = (idx & 127).reshape(_BATCH, 1)
    gathered = _sc_gather(packed, row)
    out_t = _tc_decode(gathered, lane, item_table)
    return jnp.transpose(out_t, (2, 0, 1))
